# baseline (speedup 1.0000x reference)
import jax
import jax.numpy as jnp
from jax.experimental import pallas as pl
from jax.experimental.pallas import tpu as pltpu

D = 2048
H = 256
E = 8
K = 2
B = 2
S = 2048

TM = 512
TC = 128


def _gate_kernel(x_ref, w_ref, b_ref, gated_ref, psum_ref):
    i = pl.program_id(0)
    xt = x_ref[...]
    z = jnp.dot(xt.astype(jnp.bfloat16), w_ref[...],
                preferred_element_type=jnp.float32)
    g = xt * jax.nn.sigmoid(z + b_ref[...])
    gated_ref[...] = g
    colsum = jnp.sum(g, axis=0, keepdims=True)[None]

    @pl.when(i % (S // TM) == 0)
    def _init():
        psum_ref[...] = colsum

    @pl.when(i % (S // TM) != 0)
    def _acc():
        psum_ref[...] += colsum


def _router_kernel(psum_ref, w1_ref, b1_ref, w2_ref, b2_ref,
                   probs_ref, gains_ref):
    pooled = psum_ref[...].reshape(B, D) * (1.0 / S)
    h = jnp.tanh(
        jnp.dot(pooled.astype(jnp.bfloat16), w1_ref[...],
                preferred_element_type=jnp.float32) + b1_ref[...])
    logits = (jnp.dot(h.astype(jnp.bfloat16), w2_ref[...],
                      preferred_element_type=jnp.float32) + b2_ref[...])
    ids = jax.lax.broadcasted_iota(jnp.int32, (B, E), 1)
    v1 = jnp.max(logits, axis=1, keepdims=True)
    i1 = jnp.min(jnp.where(logits == v1, ids, E), axis=1, keepdims=True)
    m1 = ids == i1
    masked = jnp.where(m1, -jnp.inf, logits)
    v2 = jnp.max(masked, axis=1, keepdims=True)
    i2 = jnp.min(jnp.where(masked == v2, ids, E), axis=1, keepdims=True)
    m2 = ids == i2
    ex = jnp.exp(logits - v1)
    probs_ref[...] = ex / jnp.sum(ex, axis=1, keepdims=True)
    e2 = jnp.exp(v2 - v1)
    w1 = 1.0 / (1.0 + e2)
    w2 = e2 * w1
    gains_ref[...] = jnp.where(m1, w1, 0.0) + jnp.where(m2, w2, 0.0)


def _bcast_kernel(gains_ref, gated_ref, out_ref):
    g = gated_ref[0]
    gv = gains_ref[0]
    for e in range(E):
        out_ref[e, 0] = g * gv[0, e]


def kernel(x, gate_W, gate_b, W1, b1, W2, b2):
    xf = x.reshape(B * S, D)
    wb = gate_W.astype(jnp.bfloat16)

    gated, psum = pl.pallas_call(
        _gate_kernel,
        grid=(B * S // TM,),
        in_specs=[
            pl.BlockSpec((TM, D), lambda i: (i, 0)),
            pl.BlockSpec((D, D), lambda i: (0, 0)),
            pl.BlockSpec((1, D), lambda i: (0, 0)),
        ],
        out_specs=[
            pl.BlockSpec((TM, D), lambda i: (i, 0)),
            pl.BlockSpec((1, 1, D), lambda i: (i // (S // TM), 0, 0)),
        ],
        out_shape=[
            jax.ShapeDtypeStruct((B * S, D), jnp.float32),
            jax.ShapeDtypeStruct((B, 1, D), jnp.float32),
        ],
        compiler_params=pltpu.CompilerParams(
            dimension_semantics=("arbitrary",)),
    )(xf, wb, gate_b.reshape(1, D))

    probs, gains = pl.pallas_call(
        _router_kernel,
        out_shape=[
            jax.ShapeDtypeStruct((B, E), jnp.float32),
            jax.ShapeDtypeStruct((B, E), jnp.float32),
        ],
    )(psum.reshape(B, D), W1.astype(jnp.bfloat16), b1.reshape(1, H),
      W2.astype(jnp.bfloat16), b2.reshape(1, E))

    routed = pl.pallas_call(
        _bcast_kernel,
        grid=(B, S // TC),
        in_specs=[
            pl.BlockSpec((1, 1, E), lambda b, i: (b, 0, 0)),
            pl.BlockSpec((1, TC, D), lambda b, i: (b, i, 0)),
        ],
        out_specs=pl.BlockSpec((E, 1, TC, D), lambda b, i: (0, b, i, 0)),
        out_shape=jax.ShapeDtypeStruct((E, B, S, D), jnp.float32),
        compiler_params=pltpu.CompilerParams(
            dimension_semantics=("parallel", "parallel")),
    )(gains.reshape(B, 1, E), gated.reshape(B, S, D))

    return routed, probs
